# baseline (speedup 1.0000x reference)
import jax
import jax.numpy as jnp
from jax.experimental import pallas as pl
from jax.experimental.pallas import tpu as pltpu

_B, _T, _R, _C = 4, 8192, 129, 2
_TT, _TI = _T // 128, 128
_D1 = _TT * _C
_NP = 20
_BL = 32


def _reg_tables(src_token):
    t = jnp.asarray(src_token, jnp.int32)
    reg = jnp.arange(_NP, dtype=jnp.int32)

    is_start = t == 0
    is_digit = (t >= 1) & (t <= 10)
    is_plus = t == 11
    is_minus = t == 12
    is_equals = t == 13
    digit_val = (t - 1) % 10

    digit_band = (reg >= 2) & (reg <= 11)
    digit_hit = reg == 2 + (digit_val % 10)
    op_reg = reg == 1
    result_regs = (reg >= 14) & (reg <= 16)

    m0 = (is_start
          | (is_digit & digit_band)
          | ((is_plus | is_minus) & op_reg)
          | (is_equals & (result_regs | op_reg | digit_band)))
    m1 = jnp.broadcast_to(is_start, (_NP,))
    v = jnp.zeros((_NP,), jnp.float32)
    v = jnp.where(is_digit & digit_hit, 1.0, v)
    v = jnp.where(is_plus & op_reg, 1.0, v)
    v = jnp.where(is_minus & op_reg, -1.0, v)
    return m0.astype(jnp.int32), m1.astype(jnp.int32), v


def _body(m0_ref, m1_ref, v_ref, x_ref, o_ref):
    col0 = (jax.lax.broadcasted_iota(jnp.int32, (_BL, _TI), 0) % _C) == 0
    for r in range(_NP):
        cond = jnp.where(col0, m0_ref[r], m1_ref[r]) != 0
        x_r = x_ref[0, r]
        o_ref[0, r] = jnp.where(cond, v_ref[r], x_r)
    o_ref[0, _NP:] = x_ref[0, _NP:]


def kernel(carrier_freq, src_token, tgt_token):
    x4 = (
        carrier_freq.transpose(0, 2, 1, 3)
        .reshape(_B, _R, _TT, _TI, _C)
        .transpose(0, 1, 2, 4, 3)
        .reshape(_B, _R, _D1, _TI)
    )
    m0, m1, v = _reg_tables(src_token)
    out = pl.pallas_call(
        _body,
        grid=(_B, _D1 // _BL),
        in_specs=[
            pl.BlockSpec(memory_space=pltpu.SMEM),
            pl.BlockSpec(memory_space=pltpu.SMEM),
            pl.BlockSpec(memory_space=pltpu.SMEM),
            pl.BlockSpec((1, _R, _BL, _TI), lambda i, j: (i, 0, j, 0)),
        ],
        out_specs=pl.BlockSpec((1, _R, _BL, _TI), lambda i, j: (i, 0, j, 0)),
        out_shape=jax.ShapeDtypeStruct((_B, _R, _D1, _TI), jnp.float32),
        compiler_params=pltpu.CompilerParams(
            dimension_semantics=("parallel", "parallel"),
        ),
    )(m0, m1, v, x4)
    return (
        out.reshape(_B, _R, _TT, _C, _TI)
        .transpose(0, 1, 2, 4, 3)
        .reshape(_B, _R, _T, _C)
        .transpose(0, 2, 1, 3)
    )


# speedup vs baseline: 1.1670x; 1.1670x over previous
import jax
import jax.numpy as jnp
from jax.experimental import pallas as pl
from jax.experimental.pallas import tpu as pltpu

_B, _T, _R, _C = 4, 8192, 129, 2
_TT, _TI = _T // 128, 128
_D1 = _TT * _C
_NP = 20
_BL = 128


def _reg_tables(src_token):
    t = jnp.asarray(src_token, jnp.int32)
    reg = jnp.arange(_NP, dtype=jnp.int32)

    is_start = t == 0
    is_digit = (t >= 1) & (t <= 10)
    is_plus = t == 11
    is_minus = t == 12
    is_equals = t == 13
    digit_val = (t - 1) % 10

    digit_band = (reg >= 2) & (reg <= 11)
    digit_hit = reg == 2 + (digit_val % 10)
    op_reg = reg == 1
    result_regs = (reg >= 14) & (reg <= 16)

    m0 = (is_start
          | (is_digit & digit_band)
          | ((is_plus | is_minus) & op_reg)
          | (is_equals & (result_regs | op_reg | digit_band)))
    m1 = jnp.broadcast_to(is_start, (_NP,))
    v = jnp.zeros((_NP,), jnp.float32)
    v = jnp.where(is_digit & digit_hit, 1.0, v)
    v = jnp.where(is_plus & op_reg, 1.0, v)
    v = jnp.where(is_minus & op_reg, -1.0, v)
    return m0.astype(jnp.int32), m1.astype(jnp.int32), v


def _body(m0_ref, m1_ref, v_ref, x_ref, o_ref):
    col0 = (jax.lax.broadcasted_iota(jnp.int32, (_BL, _TI), 0) % _C) == 0
    for r in range(_NP):
        cond = jnp.where(col0, m0_ref[r], m1_ref[r]) != 0
        x_r = x_ref[0, r]
        o_ref[0, r] = jnp.where(cond, v_ref[r], x_r)
    o_ref[0, _NP:] = x_ref[0, _NP:]


def kernel(carrier_freq, src_token, tgt_token):
    x4 = (
        carrier_freq.transpose(0, 2, 1, 3)
        .reshape(_B, _R, _TT, _TI, _C)
        .transpose(0, 1, 2, 4, 3)
        .reshape(_B, _R, _D1, _TI)
    )
    m0, m1, v = _reg_tables(src_token)
    out = pl.pallas_call(
        _body,
        grid=(_B, _D1 // _BL),
        in_specs=[
            pl.BlockSpec(memory_space=pltpu.SMEM),
            pl.BlockSpec(memory_space=pltpu.SMEM),
            pl.BlockSpec(memory_space=pltpu.SMEM),
            pl.BlockSpec((1, _R, _BL, _TI), lambda i, j: (i, 0, j, 0)),
        ],
        out_specs=pl.BlockSpec((1, _R, _BL, _TI), lambda i, j: (i, 0, j, 0)),
        out_shape=jax.ShapeDtypeStruct((_B, _R, _D1, _TI), jnp.float32),
        compiler_params=pltpu.CompilerParams(
            dimension_semantics=("parallel", "parallel"),
        ),
    )(m0, m1, v, x4)
    return (
        out.reshape(_B, _R, _TT, _C, _TI)
        .transpose(0, 1, 2, 4, 3)
        .reshape(_B, _R, _T, _C)
        .transpose(0, 2, 1, 3)
    )
